# baseline (speedup 1.0000x reference)
import functools

import jax
import jax.numpy as jnp
from jax import lax
from jax.experimental import pallas as pl
from jax.experimental.pallas import tpu as pltpu
from jax.experimental.pallas import tpu_sc as plsc

N = 10000
D = 128
E = 320000
PE_DIM = 37
PE_PAD = 64
N_EMB = 17

NC = 2
NS = 16
L = 16
NW = NC * NS

K = 88
CH = -(-E // (NW * K))
EP = NW * CH * K
S_ROWS = 10112
ROWS_PT = S_ROWS // NS

BR = 2000


def _tc_pre(x, pe, pw, pb):
    def body(x_ref, pe_ref, pw_ref, pb_ref, o_ref):
        o_ref[...] = (x_ref[...]
                      + jnp.dot(pe_ref[...], pw_ref[...],
                                preferred_element_type=jnp.float32)
                      + pb_ref[...])

    return pl.pallas_call(
        body,
        grid=(N // BR,),
        in_specs=[
            pl.BlockSpec((BR, D), lambda i: (i, 0)),
            pl.BlockSpec((BR, PE_PAD), lambda i: (i, 0)),
            pl.BlockSpec((PE_PAD, D), lambda i: (0, 0)),
            pl.BlockSpec((1, D), lambda i: (0, 0)),
        ],
        out_specs=pl.BlockSpec((BR, D), lambda i: (i, 0)),
        out_shape=jax.ShapeDtypeStruct((N, D), jnp.float32),
    )(x, pe, pw, pb)


def _tc_update(xn, sp, scale_row, w, b, pe=None, pw=None, pb=None):
    with_pe = pe is not None

    def body(*refs):
        if with_pe:
            xn_ref, sp_ref, sc_ref, w_ref, b_ref, pe_ref, pw_ref, pb_ref, o_ref = refs
        else:
            xn_ref, sp_ref, sc_ref, w_ref, b_ref, o_ref = refs
        z = sc_ref[...] * xn_ref[...] + sp_ref[0] + sp_ref[1]
        acc = jnp.dot(z, w_ref[...], preferred_element_type=jnp.float32) + b_ref[...]
        if with_pe:
            acc = acc + jnp.dot(pe_ref[...], pw_ref[...],
                                preferred_element_type=jnp.float32) + pb_ref[...]
        o_ref[...] = acc

    in_specs = [
        pl.BlockSpec((BR, D), lambda i: (i, 0)),
        pl.BlockSpec((2, BR, D), lambda i: (0, i, 0)),
        pl.BlockSpec((1, D), lambda i: (0, 0)),
        pl.BlockSpec((D, D), lambda i: (0, 0)),
        pl.BlockSpec((1, D), lambda i: (0, 0)),
    ]
    args = [xn, sp, scale_row, w, b]
    if with_pe:
        in_specs += [
            pl.BlockSpec((BR, PE_PAD), lambda i: (i, 0)),
            pl.BlockSpec((PE_PAD, D), lambda i: (0, 0)),
            pl.BlockSpec((1, D), lambda i: (0, 0)),
        ]
        args += [pe, pw, pb]

    return pl.pallas_call(
        body,
        grid=(N // BR,),
        in_specs=in_specs,
        out_specs=pl.BlockSpec((BR, D), lambda i: (i, 0)),
        out_shape=jax.ShapeDtypeStruct((N, D), jnp.float32),
    )(*args)


def _sc_agg(xn, idx4, emb):
    mesh = plsc.VectorSubcoreMesh(core_axis_name="c", subcore_axis_name="s")

    @functools.partial(
        pl.kernel,
        mesh=mesh,
        out_type=jax.ShapeDtypeStruct((NC, S_ROWS, D), jnp.float32),
        scratch_types=[
            pltpu.VMEM((3, K), jnp.int32),
            pltpu.VMEM((3, K), jnp.int32),
            pltpu.VMEM((K, D), jnp.float32),
            pltpu.VMEM((K, D), jnp.float32),
            pltpu.VMEM((K, D), jnp.float32),
            pltpu.VMEM((K, D), jnp.float32),
            pltpu.VMEM_SHARED((S_ROWS, D), jnp.float32),
            pltpu.VMEM_SHARED((N_EMB, D), jnp.float32),
            pltpu.SemaphoreType.DMA,
            pltpu.SemaphoreType.DMA,
            pltpu.SemaphoreType.DMA,
            pltpu.SemaphoreType.DMA,
            pltpu.SemaphoreType.DMA,
            pltpu.SemaphoreType.DMA,
        ],
    )
    def k(xn_h, idx_h, emb_h, out_h, s_idx0, s_idx1, xe0, xe1,
          buf0, buf1, s_acc, emb_sp,
          sem_i0, sem_i1, sem_g0, sem_g1, sem_x0, sem_x1):
        c = lax.axis_index("c")
        s = lax.axis_index("s")
        wid = s * NC + c
        slots = ((s_idx0, buf0, sem_i0, sem_g0, xe0, sem_x0),
                 (s_idx1, buf1, sem_i1, sem_g1, xe1, sem_x1))

        def idx_copy(j, slot):
            return pltpu.make_async_copy(idx_h.at[wid, j], slot[0], slot[2])

        def gather_copy(slot):
            return pltpu.make_async_copy(xn_h.at[slot[0].at[0]], slot[1], slot[3])

        def xe_copy(slot):
            return pltpu.make_async_copy(
                emb_sp.at[slot[0].at[2]], slot[4], slot[5])

        @pl.when(s == 0)
        def _():
            pltpu.sync_copy(emb_h, emb_sp)

        zero = jnp.zeros((L,), jnp.float32)

        def zbody(r, _):
            for dd in range(D // L):
                buf0[r, pl.ds(dd * L, L)] = zero
            return 0

        lax.fori_loop(0, K, zbody, 0)
        base = s * ROWS_PT
        for z in range(ROWS_PT // K):
            pltpu.sync_copy(buf0, s_acc.at[pl.ds(base + z * K, K)])
        rem = ROWS_PT % K
        if rem:
            pltpu.sync_copy(buf0.at[pl.ds(0, rem)],
                            s_acc.at[pl.ds(base + (ROWS_PT // K) * K, rem)])
        plsc.subcore_barrier()

        def compute(slot):
            s_idx, buf, xe = slot[0], slot[1], slot[4]

            def rbody(r, _):
                for dd in range(D // L):
                    sl = pl.ds(dd * L, L)
                    buf[r, sl] = jnp.maximum(buf[r, sl] + xe[r, sl], 0.0)
                return 0

            lax.fori_loop(0, K, rbody, 0)
            pltpu.sync_copy(buf, s_acc.at[s_idx.at[1]], add=True)

        idx_copy(0, slots[0]).start()
        idx_copy(1, slots[1]).start()
        idx_copy(0, slots[0]).wait()
        xe_copy(slots[0]).start()
        gather_copy(slots[0]).start()

        def mbody(jj, _):
            for b in range(2):
                j = jj * 2 + b
                sb, snb = slots[b], slots[1 - b]

                @pl.when(j + 1 < CH)
                def _():
                    idx_copy(j + 1, snb).wait()
                    xe_copy(snb).start()
                    gather_copy(snb).start()

                @pl.when(j < CH)
                def _():
                    gather_copy(sb).wait()
                    xe_copy(sb).wait()
                    compute(sb)

                @pl.when(j + 2 < CH)
                def _():
                    idx_copy(j + 2, sb).start()
            return 0

        lax.fori_loop(0, (CH + 1) // 2, mbody, 0)
        plsc.subcore_barrier()
        pltpu.sync_copy(s_acc.at[pl.ds(base, ROWS_PT)],
                        out_h.at[c, pl.ds(base, ROWS_PT)])

    return k(xn, idx4, emb)


def kernel(X_n, edge_index, edge_attr, PE, emb0, peW0, peb0, eps0, W0, b0,
           emb1, peW1, peb1, eps1, W1, b1):
    pe_p = jnp.pad(PE, ((0, 0), (0, PE_PAD - PE_DIM)))
    pw0 = jnp.pad(peW0, ((0, PE_PAD - PE_DIM), (0, 0)))
    pw1 = jnp.pad(peW1, ((0, PE_PAD - PE_DIM), (0, 0)))

    pad = EP - E
    src3 = jnp.concatenate(
        [edge_index[0], jnp.zeros((pad,), jnp.int32)]).reshape(NW, CH, 1, K)
    dst3 = jnp.concatenate(
        [edge_index[1], jnp.full((pad,), N, jnp.int32)]).reshape(NW, CH, 1, K)
    ea3 = jnp.concatenate(
        [edge_attr, jnp.zeros((pad,), jnp.int32)]).reshape(NW, CH, 1, K)
    idx4 = jnp.concatenate([src3, dst3, ea3], axis=2)

    peb0r = peb0.reshape(1, D)
    peb1r = peb1.reshape(1, D)
    b0r = b0.reshape(1, D)
    b1r = b1.reshape(1, D)
    sc0 = jnp.full((1, D), 1.0, jnp.float32) + eps0[0]
    sc1 = jnp.full((1, D), 1.0, jnp.float32) + eps1[0]

    xn1 = _tc_pre(X_n, pe_p, pw0, peb0r)
    sp1 = _sc_agg(xn1, idx4, emb0)
    xn2 = _tc_update(xn1, sp1, sc0, W0, b0r, pe_p, pw1, peb1r)
    sp2 = _sc_agg(xn2, idx4, emb1)
    return _tc_update(xn2, sp2, sc1, W1, b1r)


# speedup vs baseline: 1.0762x; 1.0762x over previous
import functools

import jax
import jax.numpy as jnp
from jax import lax
from jax.experimental import pallas as pl
from jax.experimental.pallas import tpu as pltpu
from jax.experimental.pallas import tpu_sc as plsc

N = 10000
D = 128
E = 320000
PE_DIM = 37
PE_PAD = 64
N_EMB = 17

NC = 2
NS = 16
L = 16
NW = NC * NS

K = 80
CH = -(-E // (NW * K))
EP = NW * CH * K
S_ROWS = 10112
ROWS_PT = S_ROWS // NS

BR = 2000


def _tc_pre(x, pe, pw, pb):
    def body(x_ref, pe_ref, pw_ref, pb_ref, o_ref):
        o_ref[...] = (x_ref[...]
                      + jnp.dot(pe_ref[...], pw_ref[...],
                                preferred_element_type=jnp.float32)
                      + pb_ref[...])

    return pl.pallas_call(
        body,
        grid=(N // BR,),
        in_specs=[
            pl.BlockSpec((BR, D), lambda i: (i, 0)),
            pl.BlockSpec((BR, PE_PAD), lambda i: (i, 0)),
            pl.BlockSpec((PE_PAD, D), lambda i: (0, 0)),
            pl.BlockSpec((1, D), lambda i: (0, 0)),
        ],
        out_specs=pl.BlockSpec((BR, D), lambda i: (i, 0)),
        out_shape=jax.ShapeDtypeStruct((N, D), jnp.float32),
    )(x, pe, pw, pb)


def _tc_update(xn, sp, scale_row, w, b, pe=None, pw=None, pb=None):
    with_pe = pe is not None

    def body(*refs):
        if with_pe:
            xn_ref, sp_ref, sc_ref, w_ref, b_ref, pe_ref, pw_ref, pb_ref, o_ref = refs
        else:
            xn_ref, sp_ref, sc_ref, w_ref, b_ref, o_ref = refs
        z = sc_ref[...] * xn_ref[...] + sp_ref[0] + sp_ref[1]
        acc = jnp.dot(z, w_ref[...], preferred_element_type=jnp.float32) + b_ref[...]
        if with_pe:
            acc = acc + jnp.dot(pe_ref[...], pw_ref[...],
                                preferred_element_type=jnp.float32) + pb_ref[...]
        o_ref[...] = acc

    in_specs = [
        pl.BlockSpec((BR, D), lambda i: (i, 0)),
        pl.BlockSpec((2, BR, D), lambda i: (0, i, 0)),
        pl.BlockSpec((1, D), lambda i: (0, 0)),
        pl.BlockSpec((D, D), lambda i: (0, 0)),
        pl.BlockSpec((1, D), lambda i: (0, 0)),
    ]
    args = [xn, sp, scale_row, w, b]
    if with_pe:
        in_specs += [
            pl.BlockSpec((BR, PE_PAD), lambda i: (i, 0)),
            pl.BlockSpec((PE_PAD, D), lambda i: (0, 0)),
            pl.BlockSpec((1, D), lambda i: (0, 0)),
        ]
        args += [pe, pw, pb]

    return pl.pallas_call(
        body,
        grid=(N // BR,),
        in_specs=in_specs,
        out_specs=pl.BlockSpec((BR, D), lambda i: (i, 0)),
        out_shape=jax.ShapeDtypeStruct((N, D), jnp.float32),
    )(*args)


def _sc_agg(xn, idx4, emb):
    mesh = plsc.VectorSubcoreMesh(core_axis_name="c", subcore_axis_name="s")

    @functools.partial(
        pl.kernel,
        mesh=mesh,
        out_type=jax.ShapeDtypeStruct((NC, S_ROWS, D), jnp.float32),
        scratch_types=[
            pltpu.VMEM((3, K), jnp.int32),
            pltpu.VMEM((3, K), jnp.int32),
            pltpu.VMEM((K, D), jnp.float32),
            pltpu.VMEM((K, D), jnp.float32),
            pltpu.VMEM((K, D), jnp.float32),
            pltpu.VMEM((K, D), jnp.float32),
            pltpu.VMEM_SHARED((S_ROWS, D), jnp.float32),
            pltpu.VMEM_SHARED((N_EMB, D), jnp.float32),
            pltpu.SemaphoreType.DMA,
            pltpu.SemaphoreType.DMA,
            pltpu.SemaphoreType.DMA,
            pltpu.SemaphoreType.DMA,
            pltpu.SemaphoreType.DMA,
            pltpu.SemaphoreType.DMA,
        ],
    )
    def k(xn_h, idx_h, emb_h, out_h, s_idx0, s_idx1, xe0, xe1,
          buf0, buf1, s_acc, emb_sp,
          sem_i0, sem_i1, sem_g0, sem_g1, sem_x0, sem_x1):
        c = lax.axis_index("c")
        s = lax.axis_index("s")
        wid = s * NC + c
        slots = ((s_idx0, buf0, sem_i0, sem_g0, xe0, sem_x0),
                 (s_idx1, buf1, sem_i1, sem_g1, xe1, sem_x1))

        def idx_copy(j, slot):
            return pltpu.make_async_copy(idx_h.at[wid, j], slot[0], slot[2])

        def gather_copy(slot):
            return pltpu.make_async_copy(xn_h.at[slot[0].at[0]], slot[1], slot[3])

        def xe_copy(slot):
            return pltpu.make_async_copy(
                emb_sp.at[slot[0].at[2]], slot[4], slot[5])

        @pl.when(s == 0)
        def _():
            pltpu.sync_copy(emb_h, emb_sp)

        zero = jnp.zeros((L,), jnp.float32)

        def zbody(r, _):
            for dd in range(D // L):
                buf0[r, pl.ds(dd * L, L)] = zero
            return 0

        lax.fori_loop(0, K, zbody, 0)
        base = s * ROWS_PT
        for z in range(ROWS_PT // K):
            pltpu.sync_copy(buf0, s_acc.at[pl.ds(base + z * K, K)])
        rem = ROWS_PT % K
        if rem:
            pltpu.sync_copy(buf0.at[pl.ds(0, rem)],
                            s_acc.at[pl.ds(base + (ROWS_PT // K) * K, rem)])
        plsc.subcore_barrier()

        def compute(slot):
            s_idx, buf, xe = slot[0], slot[1], slot[4]

            def rbody(r, _):
                for dd in range(D // L):
                    sl = pl.ds(dd * L, L)
                    buf[r, sl] = jnp.maximum(buf[r, sl] + xe[r, sl], 0.0)
                return 0

            lax.fori_loop(0, K, rbody, 0)
            pltpu.sync_copy(buf, s_acc.at[s_idx.at[1]], add=True)

        idx_copy(0, slots[0]).start()
        idx_copy(1, slots[1]).start()
        idx_copy(0, slots[0]).wait()
        xe_copy(slots[0]).start()
        gather_copy(slots[0]).start()

        def mbody(jj, _):
            for b in range(2):
                j = jj * 2 + b
                sb, snb = slots[b], slots[1 - b]

                @pl.when(j + 1 < CH)
                def _():
                    idx_copy(j + 1, snb).wait()
                    xe_copy(snb).start()
                    gather_copy(snb).start()

                @pl.when(j < CH)
                def _():
                    gather_copy(sb).wait()
                    xe_copy(sb).wait()
                    compute(sb)

                @pl.when(j + 2 < CH)
                def _():
                    idx_copy(j + 2, sb).start()
            return 0

        lax.fori_loop(0, (CH + 1) // 2, mbody, 0)
        plsc.subcore_barrier()
        pltpu.sync_copy(s_acc.at[pl.ds(base, ROWS_PT)],
                        out_h.at[c, pl.ds(base, ROWS_PT)])

    return k(xn, idx4, emb)


def kernel(X_n, edge_index, edge_attr, PE, emb0, peW0, peb0, eps0, W0, b0,
           emb1, peW1, peb1, eps1, W1, b1):
    pe_p = jnp.pad(PE, ((0, 0), (0, PE_PAD - PE_DIM)))
    pw0 = jnp.pad(peW0, ((0, PE_PAD - PE_DIM), (0, 0)))
    pw1 = jnp.pad(peW1, ((0, PE_PAD - PE_DIM), (0, 0)))

    pad = EP - E
    src3 = jnp.concatenate(
        [edge_index[0], jnp.zeros((pad,), jnp.int32)]).reshape(NW, CH, 1, K)
    dst3 = jnp.concatenate(
        [edge_index[1], jnp.full((pad,), N, jnp.int32)]).reshape(NW, CH, 1, K)
    ea3 = jnp.concatenate(
        [edge_attr, jnp.zeros((pad,), jnp.int32)]).reshape(NW, CH, 1, K)
    idx4 = jnp.concatenate([src3, dst3, ea3], axis=2)

    peb0r = peb0.reshape(1, D)
    peb1r = peb1.reshape(1, D)
    b0r = b0.reshape(1, D)
    b1r = b1.reshape(1, D)
    sc0 = jnp.full((1, D), 1.0, jnp.float32) + eps0[0]
    sc1 = jnp.full((1, D), 1.0, jnp.float32) + eps1[0]

    xn1 = _tc_pre(X_n, pe_p, pw0, peb0r)
    sp1 = _sc_agg(xn1, idx4, emb0)
    xn2 = _tc_update(xn1, sp1, sc0, W0, b0r, pe_p, pw1, peb1r)
    sp2 = _sc_agg(xn2, idx4, emb1)
    return _tc_update(xn2, sp2, sc1, W1, b1r)


# speedup vs baseline: 1.1978x; 1.1130x over previous
import functools

import jax
import jax.numpy as jnp
from jax import lax
from jax.experimental import pallas as pl
from jax.experimental.pallas import tpu as pltpu
from jax.experimental.pallas import tpu_sc as plsc

N = 10000
D = 128
E = 320000
PE_DIM = 37
PE_PAD = 64
N_EMB = 17

NC = 2
NS = 16
L = 16
NW = NC * NS

K = 80
CH = -(-E // (NW * K))
EP = NW * CH * K
S_ROWS = 10112
ROWS_PT = S_ROWS // NS

BR = 2000


def _tc_pre(x, pe, pw, pb):
    def body(x_ref, pe_ref, pw_ref, pb_ref, o_ref):
        o_ref[...] = (x_ref[...]
                      + jnp.dot(pe_ref[...], pw_ref[...],
                                preferred_element_type=jnp.float32)
                      + pb_ref[...])

    return pl.pallas_call(
        body,
        grid=(N // BR,),
        in_specs=[
            pl.BlockSpec((BR, D), lambda i: (i, 0)),
            pl.BlockSpec((BR, PE_PAD), lambda i: (i, 0)),
            pl.BlockSpec((PE_PAD, D), lambda i: (0, 0)),
            pl.BlockSpec((1, D), lambda i: (0, 0)),
        ],
        out_specs=pl.BlockSpec((BR, D), lambda i: (i, 0)),
        out_shape=jax.ShapeDtypeStruct((N, D), jnp.float32),
    )(x, pe, pw, pb)


def _tc_update(xn, sp, scale_row, w, b, pe=None, pw=None, pb=None):
    with_pe = pe is not None

    def body(*refs):
        if with_pe:
            xn_ref, sp_ref, sc_ref, w_ref, b_ref, pe_ref, pw_ref, pb_ref, o_ref = refs
        else:
            xn_ref, sp_ref, sc_ref, w_ref, b_ref, o_ref = refs
        z = sc_ref[...] * xn_ref[...] + sp_ref[0] + sp_ref[1]
        acc = jnp.dot(z, w_ref[...], preferred_element_type=jnp.float32) + b_ref[...]
        if with_pe:
            acc = acc + jnp.dot(pe_ref[...], pw_ref[...],
                                preferred_element_type=jnp.float32) + pb_ref[...]
        o_ref[...] = acc

    in_specs = [
        pl.BlockSpec((BR, D), lambda i: (i, 0)),
        pl.BlockSpec((2, BR, D), lambda i: (0, i, 0)),
        pl.BlockSpec((1, D), lambda i: (0, 0)),
        pl.BlockSpec((D, D), lambda i: (0, 0)),
        pl.BlockSpec((1, D), lambda i: (0, 0)),
    ]
    args = [xn, sp, scale_row, w, b]
    if with_pe:
        in_specs += [
            pl.BlockSpec((BR, PE_PAD), lambda i: (i, 0)),
            pl.BlockSpec((PE_PAD, D), lambda i: (0, 0)),
            pl.BlockSpec((1, D), lambda i: (0, 0)),
        ]
        args += [pe, pw, pb]

    return pl.pallas_call(
        body,
        grid=(N // BR,),
        in_specs=in_specs,
        out_specs=pl.BlockSpec((BR, D), lambda i: (i, 0)),
        out_shape=jax.ShapeDtypeStruct((N, D), jnp.float32),
    )(*args)


def _sc_agg(xn, idx4, dstw, emb):
    mesh = plsc.VectorSubcoreMesh(core_axis_name="c", subcore_axis_name="s")

    @functools.partial(
        pl.kernel,
        mesh=mesh,
        out_type=jax.ShapeDtypeStruct((NC, S_ROWS, D), jnp.float32),
        scratch_types=[
            pltpu.VMEM((3, K), jnp.int32),
            pltpu.VMEM((3, K), jnp.int32),
            pltpu.VMEM((K,), jnp.int32),
            pltpu.VMEM((K,), jnp.int32),
            pltpu.VMEM((K, D), jnp.float32),
            pltpu.VMEM((K, D), jnp.float32),
            pltpu.VMEM((K, D), jnp.float32),
            pltpu.VMEM((K, D), jnp.float32),
            pltpu.VMEM_SHARED((S_ROWS, D), jnp.float32),
            pltpu.VMEM_SHARED((N_EMB, D), jnp.float32),
            pltpu.SemaphoreType.DMA,
            pltpu.SemaphoreType.DMA,
            pltpu.SemaphoreType.DMA,
            pltpu.SemaphoreType.DMA,
            pltpu.SemaphoreType.DMA,
            pltpu.SemaphoreType.DMA,
            pltpu.SemaphoreType.DMA,
            pltpu.SemaphoreType.DMA,
            pltpu.SemaphoreType.DMA,
            pltpu.SemaphoreType.DMA,
        ],
    )
    def k(xn_h, idx_h, dst_h, emb_h, out_h, s_idx0, s_idx1, sdst0, sdst1,
          xe0, xe1, buf0, buf1, s_acc, emb_sp,
          sem_i0, sem_i1, sem_g0, sem_g1, sem_x0, sem_x1,
          sem_d0, sem_d1, sem_s0, sem_s1):
        c = lax.axis_index("c")
        s = lax.axis_index("s")
        wid = s * NC + c
        slots = ((s_idx0, buf0, sem_i0, sem_g0, xe0, sem_x0, sdst0, sem_d0, sem_s0),
                 (s_idx1, buf1, sem_i1, sem_g1, xe1, sem_x1, sdst1, sem_d1, sem_s1))

        def idx_copy(j, slot):
            return pltpu.make_async_copy(idx_h.at[wid, j], slot[0], slot[2])

        def dst_copy(j, slot):
            return pltpu.make_async_copy(dst_h.at[wid, j], slot[6], slot[7])

        def scatter_start(slot):
            pltpu.async_copy(slot[1], s_acc.at[slot[6]], slot[8], add=True)

        def scatter_wait(slot):
            pltpu.make_async_copy(slot[1], s_acc.at[slot[6]], slot[8]).wait()

        def gather_copy(slot):
            return pltpu.make_async_copy(xn_h.at[slot[0].at[0]], slot[1], slot[3])

        def xe_copy(slot):
            return pltpu.make_async_copy(
                emb_sp.at[slot[0].at[2]], slot[4], slot[5])

        @pl.when(s == 0)
        def _():
            pltpu.sync_copy(emb_h, emb_sp)

        zero = jnp.zeros((L,), jnp.float32)

        def zbody(r, _):
            for dd in range(D // L):
                buf0[r, pl.ds(dd * L, L)] = zero
            return 0

        lax.fori_loop(0, K, zbody, 0)
        base = s * ROWS_PT
        for z in range(ROWS_PT // K):
            pltpu.sync_copy(buf0, s_acc.at[pl.ds(base + z * K, K)])
        rem = ROWS_PT % K
        if rem:
            pltpu.sync_copy(buf0.at[pl.ds(0, rem)],
                            s_acc.at[pl.ds(base + (ROWS_PT // K) * K, rem)])
        plsc.subcore_barrier()

        def compute(slot):
            s_idx, buf, xe = slot[0], slot[1], slot[4]

            def rbody(r, _):
                for dd in range(D // L):
                    sl = pl.ds(dd * L, L)
                    buf[r, sl] = jnp.maximum(buf[r, sl] + xe[r, sl], 0.0)
                return 0

            lax.fori_loop(0, K, rbody, 0)

        idx_copy(0, slots[0]).start()
        idx_copy(1, slots[1]).start()
        dst_copy(0, slots[0]).start()
        dst_copy(1, slots[1]).start()
        idx_copy(0, slots[0]).wait()
        xe_copy(slots[0]).start()
        gather_copy(slots[0]).start()

        def mbody(jj, _):
            for b in range(2):
                j = jj * 2 + b
                sb, snb = slots[b], slots[1 - b]

                @pl.when(j + 1 < CH)
                def _():
                    idx_copy(j + 1, snb).wait()

                    @pl.when(j >= 1)
                    def _():
                        scatter_wait(snb)
                        dst_copy(j + 1, snb).start()

                    xe_copy(snb).start()
                    gather_copy(snb).start()

                @pl.when(j < CH)
                def _():
                    gather_copy(sb).wait()
                    xe_copy(sb).wait()
                    dst_copy(j, sb).wait()
                    compute(sb)
                    scatter_start(sb)

                @pl.when(j + 2 < CH)
                def _():
                    idx_copy(j + 2, sb).start()
            return 0

        lax.fori_loop(0, (CH + 1) // 2, mbody, 0)
        scatter_wait(slots[(CH - 1) % 2])
        scatter_wait(slots[(CH - 2) % 2])
        plsc.subcore_barrier()
        pltpu.sync_copy(s_acc.at[pl.ds(base, ROWS_PT)],
                        out_h.at[c, pl.ds(base, ROWS_PT)])

    return k(xn, idx4, dstw, emb)


def kernel(X_n, edge_index, edge_attr, PE, emb0, peW0, peb0, eps0, W0, b0,
           emb1, peW1, peb1, eps1, W1, b1):
    pe_p = jnp.pad(PE, ((0, 0), (0, PE_PAD - PE_DIM)))
    pw0 = jnp.pad(peW0, ((0, PE_PAD - PE_DIM), (0, 0)))
    pw1 = jnp.pad(peW1, ((0, PE_PAD - PE_DIM), (0, 0)))

    pad = EP - E
    src3 = jnp.concatenate(
        [edge_index[0], jnp.zeros((pad,), jnp.int32)]).reshape(NW, CH, 1, K)
    dst3 = jnp.concatenate(
        [edge_index[1], jnp.full((pad,), N, jnp.int32)]).reshape(NW, CH, 1, K)
    ea3 = jnp.concatenate(
        [edge_attr, jnp.zeros((pad,), jnp.int32)]).reshape(NW, CH, 1, K)
    idx4 = jnp.concatenate([src3, dst3, ea3], axis=2)
    dstw = dst3.reshape(NW, CH, K)

    peb0r = peb0.reshape(1, D)
    peb1r = peb1.reshape(1, D)
    b0r = b0.reshape(1, D)
    b1r = b1.reshape(1, D)
    sc0 = jnp.full((1, D), 1.0, jnp.float32) + eps0[0]
    sc1 = jnp.full((1, D), 1.0, jnp.float32) + eps1[0]

    xn1 = _tc_pre(X_n, pe_p, pw0, peb0r)
    sp1 = _sc_agg(xn1, idx4, dstw, emb0)
    xn2 = _tc_update(xn1, sp1, sc0, W0, b0r, pe_p, pw1, peb1r)
    sp2 = _sc_agg(xn2, idx4, dstw, emb1)
    return _tc_update(xn2, sp2, sc1, W1, b1r)
